# Initial kernel scaffold; baseline (speedup 1.0000x reference)
import jax
import jax.numpy as jnp
from jax.experimental import pallas as pl


def kernel(x, basis, comp, root, bias):
    raise NotImplementedError("write your pallas kernel here")


# speedup vs baseline: 551.2166x; 551.2166x over previous
import functools
import jax
import jax.numpy as jnp
from jax import lax
from jax.experimental import pallas as pl
from jax.experimental.pallas import tpu as pltpu

_K = 4
_NUM_REL = _K + 1
_NUM_BASES = 2
_D = 256
_SLEN = 512


def _shift_down(a, k):
    rolled = pltpu.roll(a, k, 0)
    t = lax.broadcasted_iota(jnp.int32, a.shape, 0)
    return jnp.where(t >= k, rolled, 0.0)


def _shift_up(a, k):
    rolled = pltpu.roll(a, a.shape[0] - k, 0)
    t = lax.broadcasted_iota(jnp.int32, a.shape, 0)
    return jnp.where(t < a.shape[0] - k, rolled, 0.0)


def _rgcn_body(comp_ref, x_ref, basis_ref, root_ref, bias_ref, out_ref):
    xi = x_ref[0]

    s0 = _shift_up(xi, 1)
    s1 = xi + _shift_up(xi, 2)
    s2 = _shift_down(xi, 1) + _shift_up(xi, 3)
    s3 = _shift_down(xi, 2) + _shift_up(xi, 4)
    total = jnp.sum(xi, axis=0, keepdims=True)
    s4 = total - (s0 + s1 + s2 + s3)

    t = lax.broadcasted_iota(jnp.int32, (_SLEN, 1), 0)
    one = jnp.ones((_SLEN, 1), jnp.float32)
    zero = jnp.zeros((_SLEN, 1), jnp.float32)
    c0 = jnp.where(t <= _SLEN - 2, one, zero)
    c1 = one + jnp.where(t <= _SLEN - 3, one, zero)
    c2 = jnp.where(t >= 1, one, zero) + jnp.where(t <= _SLEN - 4, one, zero)
    c3 = jnp.where(t >= 2, one, zero) + jnp.where(t <= _SLEN - 5, one, zero)
    c4 = float(_SLEN) - (c0 + c1 + c2 + c3)

    m0 = s0 * (one / jnp.maximum(c0, 1.0))
    m1 = s1 * (one / c1)
    m2 = s2 * (one / jnp.maximum(c2, 1.0))
    m3 = s3 * (one / jnp.maximum(c3, 1.0))
    m4 = s4 * (one / c4)

    acc = xi @ root_ref[...]
    for b in range(_NUM_BASES):
        mb = (comp_ref[0, b] * m0 + comp_ref[1, b] * m1
              + comp_ref[2, b] * m2 + comp_ref[3, b] * m3
              + comp_ref[4, b] * m4)
        acc = acc + jnp.dot(mb, basis_ref[b],
                            preferred_element_type=jnp.float32)
    out_ref[0] = acc + bias_ref[...]


@jax.jit
def kernel(x, basis, comp, root, bias):
    batch = x.shape[0]
    grid_spec = pltpu.PrefetchScalarGridSpec(
        num_scalar_prefetch=1,
        grid=(batch,),
        in_specs=[
            pl.BlockSpec((1, _SLEN, _D), lambda i, c: (i, 0, 0)),
            pl.BlockSpec((_NUM_BASES, _D, _D), lambda i, c: (0, 0, 0)),
            pl.BlockSpec((_D, _D), lambda i, c: (0, 0)),
            pl.BlockSpec((1, _D), lambda i, c: (0, 0)),
        ],
        out_specs=pl.BlockSpec((1, _SLEN, _D), lambda i, c: (i, 0, 0)),
    )
    out = pl.pallas_call(
        _rgcn_body,
        grid_spec=grid_spec,
        out_shape=jax.ShapeDtypeStruct((batch, _SLEN, _D), jnp.float32),
    )(comp, x, basis, root, bias.reshape(1, _D))
    return out
